# initial kernel scaffold (unmeasured)
import jax
import jax.numpy as jnp
from jax import lax
from jax.experimental import pallas as pl
from jax.experimental.pallas import tpu as pltpu

N_DEV = 8
B = 4
SQ = 256
D_MODEL = 1024
H_LOC = 8
DH = 128
SKV = 1024
SCALE = 0.08838834764831843

ROWS = B * SQ
CHUNK = ROWS // N_DEV
N_HOPS = 2 * (N_DEV - 1)


def kernel(x, Wq, Wo, K_ext, V_ext):
    i = lax.axis_index("i")
    K_loc = jnp.transpose(
        lax.dynamic_slice_in_dim(K_ext, i * H_LOC, H_LOC, axis=2), (0, 2, 1, 3)
    )
    V_loc = jnp.transpose(
        lax.dynamic_slice_in_dim(V_ext, i * H_LOC, H_LOC, axis=2), (0, 2, 1, 3)
    )

    def body(x_ref, wq_ref, wo_ref, k_ref, v_ref, out_ref,
             acc_ref, comm_ref, send_sems, recv_sems):
        my = lax.axis_index("i")
        left = (my - 1) % N_DEV
        right = (my + 1) % N_DEV

        barrier_sem = pltpu.get_barrier_semaphore()
        for nbr in (left, right):
            pl.semaphore_signal(barrier_sem, inc=1, device_id=(nbr,),
                                device_id_type=pl.DeviceIdType.MESH)
        pl.semaphore_wait(barrier_sem, 2)

        wo = wo_ref[...]
        for b in range(B):
            q_b = jnp.dot(x_ref[b], wq_ref[...],
                          preferred_element_type=jnp.float32)
            heads = []
            for h in range(H_LOC):
                q_h = q_b[:, h * DH:(h + 1) * DH]
                s = lax.dot_general(
                    q_h, k_ref[b, h], (((1,), (1,)), ((), ())),
                    preferred_element_type=jnp.float32) * SCALE
                m = jnp.max(s, axis=1, keepdims=True)
                p = jnp.exp(s - m)
                l = jnp.sum(p, axis=1, keepdims=True)
                o = jnp.dot(p, v_ref[b, h],
                            preferred_element_type=jnp.float32)
                heads.append(o / l)
            attn_b = jnp.concatenate(heads, axis=1)
            acc_ref[pl.ds(b * SQ, SQ), :] = jnp.dot(
                attn_b, wo, preferred_element_type=jnp.float32)

        for s in range(N_DEV - 1):
            src_c = (my - s) % N_DEV
            rdma = pltpu.make_async_remote_copy(
                src_ref=acc_ref.at[pl.ds(src_c * CHUNK, CHUNK), :],
                dst_ref=comm_ref.at[s],
                send_sem=send_sems.at[s],
                recv_sem=recv_sems.at[s],
                device_id=(right,),
                device_id_type=pl.DeviceIdType.MESH,
            )
            rdma.start()
            rdma.wait()
            dst = pl.ds(((my - s - 1) % N_DEV) * CHUNK, CHUNK)
            acc_ref[dst, :] = acc_ref[dst, :] + comm_ref[s]

        for t in range(N_DEV - 1):
            src_c = (my + 1 - t) % N_DEV
            slot = (N_DEV - 1) + t
            rdma = pltpu.make_async_remote_copy(
                src_ref=acc_ref.at[pl.ds(src_c * CHUNK, CHUNK), :],
                dst_ref=comm_ref.at[slot],
                send_sem=send_sems.at[slot],
                recv_sem=recv_sems.at[slot],
                device_id=(right,),
                device_id_type=pl.DeviceIdType.MESH,
            )
            rdma.start()
            rdma.wait()
            acc_ref[pl.ds(((my - t) % N_DEV) * CHUNK, CHUNK), :] = comm_ref[slot]

        for b in range(B):
            out_ref[b] = acc_ref[pl.ds(b * SQ, SQ), :]

    return pl.pallas_call(
        body,
        out_shape=jax.ShapeDtypeStruct((B, SQ, D_MODEL), jnp.float32),
        in_specs=[pl.BlockSpec(memory_space=pltpu.VMEM)] * 5,
        out_specs=pl.BlockSpec(memory_space=pltpu.VMEM),
        scratch_shapes=[
            pltpu.VMEM((ROWS, D_MODEL), jnp.float32),
            pltpu.VMEM((N_HOPS, CHUNK, D_MODEL), jnp.float32),
            pltpu.SemaphoreType.DMA((N_HOPS,)),
            pltpu.SemaphoreType.DMA((N_HOPS,)),
        ],
        compiler_params=pltpu.CompilerParams(collective_id=0),
    )(x, Wq, Wo, K_loc, V_loc)


# baseline (device time: 181504 ns/iter reference)
import jax
import jax.numpy as jnp
from jax import lax
from jax.experimental import pallas as pl
from jax.experimental.pallas import tpu as pltpu

N_DEV = 8
B = 4
SQ = 256
D_MODEL = 1024
H_LOC = 8
DH = 128
SKV = 1024
SCALE = 0.08838834764831843

ROWS = B * SQ
CHUNK = ROWS // N_DEV
N_HOPS = 2 * (N_DEV - 1)


def kernel(x, Wq, Wo, K_ext, V_ext):
    i = lax.axis_index("i")
    K_loc = jnp.transpose(
        lax.dynamic_slice_in_dim(K_ext, i * H_LOC, H_LOC, axis=2), (0, 2, 1, 3)
    ).astype(jnp.bfloat16)
    V_loc = jnp.transpose(
        lax.dynamic_slice_in_dim(V_ext, i * H_LOC, H_LOC, axis=2), (0, 2, 1, 3)
    ).astype(jnp.bfloat16)
    x = x.astype(jnp.bfloat16)
    Wq = Wq.astype(jnp.bfloat16)
    Wo = Wo.astype(jnp.bfloat16)

    def body(x_ref, wq_ref, wo_ref, k_ref, v_ref, out_ref,
             acc_ref, comm_ref, send_sems, recv_sems):
        my = lax.axis_index("i")
        left = (my - 1) % N_DEV
        right = (my + 1) % N_DEV

        barrier_sem = pltpu.get_barrier_semaphore()
        for nbr in (left, right):
            pl.semaphore_signal(barrier_sem, inc=1, device_id=(nbr,),
                                device_id_type=pl.DeviceIdType.MESH)
        pl.semaphore_wait(barrier_sem, 2)

        wo = wo_ref[...]
        for b in range(B):
            q_b = jnp.dot(x_ref[b], wq_ref[...],
                          preferred_element_type=jnp.float32)
            q_b = q_b.astype(jnp.bfloat16)
            heads = []
            for h in range(H_LOC):
                q_h = q_b[:, h * DH:(h + 1) * DH]
                s = lax.dot_general(
                    q_h, k_ref[b, h], (((1,), (1,)), ((), ())),
                    preferred_element_type=jnp.float32) * SCALE
                m = jnp.max(s, axis=1, keepdims=True)
                p = jnp.exp(s - m)
                l = jnp.sum(p, axis=1, keepdims=True)
                o = jnp.dot(p.astype(jnp.bfloat16), v_ref[b, h],
                            preferred_element_type=jnp.float32)
                heads.append(o / l)
            attn_b = jnp.concatenate(heads, axis=1).astype(jnp.bfloat16)
            acc_ref[pl.ds(b * SQ, SQ), :] = jnp.dot(
                attn_b, wo, preferred_element_type=jnp.float32)

        for s in range(N_DEV - 1):
            src_c = (my - s) % N_DEV
            rdma = pltpu.make_async_remote_copy(
                src_ref=acc_ref.at[pl.ds(src_c * CHUNK, CHUNK), :],
                dst_ref=comm_ref.at[s],
                send_sem=send_sems.at[s],
                recv_sem=recv_sems.at[s],
                device_id=(right,),
                device_id_type=pl.DeviceIdType.MESH,
            )
            rdma.start()
            rdma.wait()
            dst = pl.ds(((my - s - 1) % N_DEV) * CHUNK, CHUNK)
            acc_ref[dst, :] = acc_ref[dst, :] + comm_ref[s]

        for t in range(N_DEV - 1):
            src_c = (my + 1 - t) % N_DEV
            slot = (N_DEV - 1) + t
            rdma = pltpu.make_async_remote_copy(
                src_ref=acc_ref.at[pl.ds(src_c * CHUNK, CHUNK), :],
                dst_ref=comm_ref.at[slot],
                send_sem=send_sems.at[slot],
                recv_sem=recv_sems.at[slot],
                device_id=(right,),
                device_id_type=pl.DeviceIdType.MESH,
            )
            rdma.start()
            rdma.wait()
            acc_ref[pl.ds(((my - t) % N_DEV) * CHUNK, CHUNK), :] = comm_ref[slot]

        for b in range(B):
            out_ref[b] = acc_ref[pl.ds(b * SQ, SQ), :]

    return pl.pallas_call(
        body,
        out_shape=jax.ShapeDtypeStruct((B, SQ, D_MODEL), jnp.float32),
        in_specs=[pl.BlockSpec(memory_space=pltpu.VMEM)] * 5,
        out_specs=pl.BlockSpec(memory_space=pltpu.VMEM),
        scratch_shapes=[
            pltpu.VMEM((ROWS, D_MODEL), jnp.float32),
            pltpu.VMEM((N_HOPS, CHUNK, D_MODEL), jnp.float32),
            pltpu.SemaphoreType.DMA((N_HOPS,)),
            pltpu.SemaphoreType.DMA((N_HOPS,)),
        ],
        compiler_params=pltpu.CompilerParams(
            collective_id=0, vmem_limit_bytes=100 * 1024 * 1024
        ),
    )(x, Wq, Wo, K_loc, V_loc)
